# Initial kernel scaffold; baseline (speedup 1.0000x reference)
import jax
import jax.numpy as jnp
from jax.experimental import pallas as pl


def kernel(adj_t, x, W0, b0, g0, be0, W1, b1, g1, be1, W2, b2, g2, be2, lin1_W, lin1_b):
    raise NotImplementedError("write your pallas kernel here")


# speedup vs baseline: 4.3247x; 4.3247x over previous
import functools

import jax
import jax.numpy as jnp
from jax import lax
from jax.experimental import pallas as pl
from jax.experimental.pallas import tpu as pltpu
from jax.experimental.pallas import tpu_sc as plsc

N = 10000
E = 320000
D = 128
EPS = 1e-5

NC = 2
NS = 16
NT = NC * NS
K = 128
NPAD = 10240
EPAD = 327680
EPT = EPAD // NT
NCH = EPT // K
RPT = NPAD // NS


@functools.cache
def _build_deg():
    mesh = plsc.VectorSubcoreMesh(core_axis_name="c", subcore_axis_name="s",
                                  num_cores=NC, num_subcores=NS)
    return functools.partial(
        pl.kernel,
        out_type=jax.ShapeDtypeStruct((NC, NPAD, D), jnp.float32),
        mesh=mesh,
        scratch_types=[
            pltpu.VMEM((2 * NCH, K), jnp.int32),
            pltpu.VMEM((K, D), jnp.float32),
            pltpu.VMEM_SHARED((NPAD, D), jnp.float32),
        ],
    )(_deg_body)


def _deg_body(il_hbm, e01_hbm, zeros_hbm, out_hbm, il_v, e01_v, acc_sh):
    c = lax.axis_index("c")
    s = lax.axis_index("s")
    wid = s * NC + c
    pltpu.sync_copy(il_hbm.at[wid], il_v)
    pltpu.sync_copy(e01_hbm, e01_v)
    pltpu.sync_copy(zeros_hbm.at[pl.ds(s * RPT, RPT)],
                    acc_sh.at[pl.ds(s * RPT, RPT)])
    plsc.subcore_barrier()

    def body(j, carry):
        pltpu.sync_copy(e01_v, acc_sh.at[il_v.at[j]], add=True)
        return carry

    lax.fori_loop(0, 2 * NCH, body, 0)
    plsc.subcore_barrier()
    pltpu.sync_copy(acc_sh.at[pl.ds(s * RPT, RPT)],
                    out_hbm.at[c, pl.ds(s * RPT, RPT)])


@functools.cache
def _build_agg():
    mesh = plsc.VectorSubcoreMesh(core_axis_name="c", subcore_axis_name="s",
                                  num_cores=NC, num_subcores=NS)
    return functools.partial(
        pl.kernel,
        out_type=jax.ShapeDtypeStruct((NC, NPAD, D), jnp.float32),
        mesh=mesh,
        scratch_types=[
            pltpu.VMEM((NCH, K), jnp.int32),
            pltpu.VMEM((NCH, K), jnp.int32),
            pltpu.VMEM((K, D), jnp.float32),
            pltpu.VMEM_SHARED((NPAD, D), jnp.float32),
            pltpu.SemaphoreType.DMA,
        ],
    )(_agg_body)


def _agg_body(hp_hbm, src_hbm, dst_hbm, zeros_hbm, out_hbm,
              src_v, dst_v, rows_v, acc_sh, sem):
    c = lax.axis_index("c")
    s = lax.axis_index("s")
    wid = s * NC + c
    pltpu.sync_copy(src_hbm.at[wid], src_v)
    pltpu.sync_copy(dst_hbm.at[wid], dst_v)
    pltpu.sync_copy(zeros_hbm.at[pl.ds(s * RPT, RPT)],
                    acc_sh.at[pl.ds(s * RPT, RPT)])
    plsc.subcore_barrier()

    def body(j, carry):
        pltpu.async_copy(hp_hbm.at[src_v.at[j]], rows_v, sem).wait()
        pltpu.sync_copy(rows_v, acc_sh.at[dst_v.at[j]], add=True)
        return carry

    lax.fori_loop(0, NCH, body, 0)
    plsc.subcore_barrier()
    pltpu.sync_copy(acc_sh.at[pl.ds(s * RPT, RPT)],
                    out_hbm.at[c, pl.ds(s * RPT, RPT)])


BM = 512
BML = 1000


def _norms(degq, col, row0, nrows):
    deg = degq[:, col:col + 1] + degq[:, col + 2:col + 3]
    rows = row0 + lax.broadcasted_iota(jnp.int32, (nrows, 1), 0)
    ok = (deg > 0) & (rows < N)
    return jnp.where(ok, lax.rsqrt(jnp.maximum(deg, 1.0)), 0.0)


def _pre_body(x_ref, w_ref, dq_ref, o_ref):
    m = pl.program_id(0)
    nsrc = _norms(dq_ref[...], 0, m * BM, BM)
    h = jnp.dot(x_ref[...], w_ref[...], preferred_element_type=jnp.float32)
    o_ref[...] = h * nsrc


def _tc_pre(x_pad, W, degq):
    return pl.pallas_call(
        _pre_body,
        grid=(NPAD // BM,),
        in_specs=[
            pl.BlockSpec((BM, D), lambda m: (m, 0)),
            pl.BlockSpec((D, D), lambda m: (0, 0)),
            pl.BlockSpec((BM, 4), lambda m: (m, 0)),
        ],
        out_specs=pl.BlockSpec((BM, D), lambda m: (m, 0)),
        out_shape=jax.ShapeDtypeStruct((NPAD, D), jnp.float32),
    )(x_pad, W, degq)


def _layer_h(p_ref, dq, b_ref, g_ref, be_ref, row0, nrows):
    ndst = _norms(dq, 1, row0, nrows)
    agg = (p_ref[0] + p_ref[1]) * ndst
    a1 = jnp.maximum(agg + b_ref[...], 0.0)
    gs = g_ref[...] * lax.rsqrt(jnp.float32(1.0 + EPS))
    return jnp.maximum(a1 * gs + be_ref[...], 0.0)


def _mid_body(p_ref, dq_ref, b_ref, g_ref, be_ref, w_ref,
              h_ref, hp_ref):
    m = pl.program_id(0)
    dq = dq_ref[...]
    h = _layer_h(p_ref, dq, b_ref, g_ref, be_ref, m * BM, BM)
    h_ref[...] = h
    nsrc = _norms(dq, 0, m * BM, BM)
    hp_ref[...] = jnp.dot(h, w_ref[...],
                          preferred_element_type=jnp.float32) * nsrc


def _tc_mid(partials, degq, b, g, be, Wnext):
    return pl.pallas_call(
        _mid_body,
        grid=(NPAD // BM,),
        in_specs=[
            pl.BlockSpec((NC, BM, D), lambda m: (0, m, 0)),
            pl.BlockSpec((BM, 4), lambda m: (m, 0)),
            pl.BlockSpec((1, D), lambda m: (0, 0)),
            pl.BlockSpec((1, D), lambda m: (0, 0)),
            pl.BlockSpec((1, D), lambda m: (0, 0)),
            pl.BlockSpec((D, D), lambda m: (0, 0)),
        ],
        out_specs=[
            pl.BlockSpec((BM, D), lambda m: (m, 0)),
            pl.BlockSpec((BM, D), lambda m: (m, 0)),
        ],
        out_shape=[
            jax.ShapeDtypeStruct((NPAD, D), jnp.float32),
            jax.ShapeDtypeStruct((NPAD, D), jnp.float32),
        ],
    )(partials, degq, b, g, be, Wnext)


def _last_body(p_ref, dq_ref, b_ref, g_ref, be_ref, h0_ref, h1_ref,
               lw_ref, lb_ref, o_ref):
    m = pl.program_id(0)
    h2 = _layer_h(p_ref, dq_ref[...], b_ref, g_ref, be_ref, m * BML, BML)
    jk = jnp.maximum(jnp.maximum(h0_ref[...], h1_ref[...]), h2)
    o_ref[...] = jnp.maximum(
        jnp.dot(jk, lw_ref[...], preferred_element_type=jnp.float32)
        + lb_ref[...], 0.0)


def _tc_last(partials, degq, b, g, be, h0, h1, lin1_W, lin1_b):
    return pl.pallas_call(
        _last_body,
        grid=(N // BML,),
        in_specs=[
            pl.BlockSpec((NC, BML, D), lambda m: (0, m, 0)),
            pl.BlockSpec((BML, 4), lambda m: (m, 0)),
            pl.BlockSpec((1, D), lambda m: (0, 0)),
            pl.BlockSpec((1, D), lambda m: (0, 0)),
            pl.BlockSpec((1, D), lambda m: (0, 0)),
            pl.BlockSpec((BML, D), lambda m: (m, 0)),
            pl.BlockSpec((BML, D), lambda m: (m, 0)),
            pl.BlockSpec((D, D), lambda m: (0, 0)),
            pl.BlockSpec((1, D), lambda m: (0, 0)),
        ],
        out_specs=pl.BlockSpec((BML, D), lambda m: (m, 0)),
        out_shape=jax.ShapeDtypeStruct((N, D), jnp.float32),
    )(partials, degq, b, g, be, h0, h1, lin1_W, lin1_b)


def kernel(adj_t, x, W0, b0, g0, be0, W1, b1, g1, be1, W2, b2, g2, be2,
           lin1_W, lin1_b):
    pad = EPAD - E
    padv = jnp.full((pad,), N, jnp.int32)
    srcp = jnp.concatenate([adj_t[0], padv])
    dstp = jnp.concatenate([adj_t[1], padv])
    src3 = srcp.reshape(NT, NCH, K)
    dst3 = dstp.reshape(NT, NCH, K)
    x_pad = jnp.pad(x, ((0, NPAD - N), (0, 0)))
    zeros = jnp.zeros((NPAD, D), jnp.float32)
    il3 = jnp.stack([srcp.reshape(NT, EPT), dstp.reshape(NT, EPT)],
                    axis=-1).reshape(NT, 2 * NCH, K)
    eye2 = jnp.eye(2, D, dtype=jnp.float32)
    e01 = jnp.tile(eye2, (K // 2, 1))

    degp = _build_deg()(il3, e01, zeros)
    degq = degp[:, :, :2].transpose(1, 0, 2).reshape(NPAD, NC * 2)

    b = [b0.reshape(1, D), b1.reshape(1, D), b2.reshape(1, D)]
    g = [g0.reshape(1, D), g1.reshape(1, D), g2.reshape(1, D)]
    be = [be0.reshape(1, D), be1.reshape(1, D), be2.reshape(1, D)]

    hp = _tc_pre(x_pad, W0, degq)
    p0 = _build_agg()(hp, src3, dst3, zeros)
    h0, hp = _tc_mid(p0, degq, b[0], g[0], be[0], W1)
    p1 = _build_agg()(hp, src3, dst3, zeros)
    h1, hp = _tc_mid(p1, degq, b[1], g[1], be[1], W2)
    p2 = _build_agg()(hp, src3, dst3, zeros)
    out = _tc_last(p2, degq, b[2], g[2], be[2], h0, h1,
                   lin1_W, lin1_b.reshape(1, D))
    return out


# speedup vs baseline: 5.3522x; 1.2376x over previous
import functools

import jax
import jax.numpy as jnp
from jax import lax
from jax.experimental import pallas as pl
from jax.experimental.pallas import tpu as pltpu
from jax.experimental.pallas import tpu_sc as plsc

N = 10000
E = 320000
D = 128
EPS = 1e-5

NC = 2
NS = 16
NT = NC * NS
K = 128
NPAD = 10240
EPAD = 327680
EPT = EPAD // NT
NCH = EPT // K
RPT = NPAD // NS


@functools.cache
def _build_deg():
    mesh = plsc.VectorSubcoreMesh(core_axis_name="c", subcore_axis_name="s",
                                  num_cores=NC, num_subcores=NS)
    return functools.partial(
        pl.kernel,
        out_type=jax.ShapeDtypeStruct((NC, NPAD, D), jnp.float32),
        mesh=mesh,
        scratch_types=[
            pltpu.VMEM((2 * NCH, K), jnp.int32),
            pltpu.VMEM((K, D), jnp.float32),
            pltpu.VMEM_SHARED((NPAD, D), jnp.float32),
        ],
    )(_deg_body)


def _deg_body(il_hbm, e01_hbm, zeros_hbm, out_hbm, il_v, e01_v, acc_sh):
    c = lax.axis_index("c")
    s = lax.axis_index("s")
    wid = s * NC + c
    pltpu.sync_copy(il_hbm.at[wid], il_v)
    pltpu.sync_copy(e01_hbm, e01_v)
    pltpu.sync_copy(zeros_hbm.at[pl.ds(s * RPT, RPT)],
                    acc_sh.at[pl.ds(s * RPT, RPT)])
    plsc.subcore_barrier()

    def body(j, carry):
        pltpu.sync_copy(e01_v, acc_sh.at[il_v.at[j]], add=True)
        return carry

    lax.fori_loop(0, 2 * NCH, body, 0)
    plsc.subcore_barrier()
    pltpu.sync_copy(acc_sh.at[pl.ds(s * RPT, RPT)],
                    out_hbm.at[c, pl.ds(s * RPT, RPT)])


FAST_CORE = 0
CF = 116
CS = 44


@functools.cache
def _build_agg():
    mesh = plsc.VectorSubcoreMesh(core_axis_name="c", subcore_axis_name="s",
                                  num_cores=NC, num_subcores=NS)
    return functools.partial(
        pl.kernel,
        out_type=jax.ShapeDtypeStruct((NC, NPAD, D), jnp.float32),
        mesh=mesh,
        scratch_types=[
            pltpu.VMEM((CF, K), jnp.int32),
            pltpu.VMEM((CF, K), jnp.int32),
            pltpu.VMEM((K, D), jnp.float32),
            pltpu.VMEM_SHARED((NPAD, D), jnp.float32),
            pltpu.SemaphoreType.DMA,
        ],
    )(_agg_body)


def _agg_body(hp_hbm, srcf_hbm, dstf_hbm, srcs_hbm, dsts_hbm, zeros_hbm,
              out_hbm, src_v, dst_v, rows_v, acc_sh, sem):
    c = lax.axis_index("c")
    s = lax.axis_index("s")

    @pl.when(c == FAST_CORE)
    def _():
        pltpu.sync_copy(srcf_hbm.at[s], src_v)
        pltpu.sync_copy(dstf_hbm.at[s], dst_v)

    @pl.when(c != FAST_CORE)
    def _():
        pltpu.sync_copy(srcs_hbm.at[s], src_v.at[pl.ds(0, CS)])
        pltpu.sync_copy(dsts_hbm.at[s], dst_v.at[pl.ds(0, CS)])

    pltpu.sync_copy(zeros_hbm.at[pl.ds(s * RPT, RPT)],
                    acc_sh.at[pl.ds(s * RPT, RPT)])
    plsc.subcore_barrier()

    nchunks = jnp.where(c == FAST_CORE, CF, CS)

    def body(j, carry):
        pltpu.async_copy(hp_hbm.at[src_v.at[j]], rows_v, sem).wait()
        pltpu.sync_copy(rows_v, acc_sh.at[dst_v.at[j]], add=True)
        return carry

    lax.fori_loop(0, nchunks, body, 0)
    plsc.subcore_barrier()
    pltpu.sync_copy(acc_sh.at[pl.ds(s * RPT, RPT)],
                    out_hbm.at[c, pl.ds(s * RPT, RPT)])


BM = 512
BML = 1000


def _norms(degq, col, row0, nrows):
    deg = degq[:, col:col + 1] + degq[:, col + 2:col + 3]
    rows = row0 + lax.broadcasted_iota(jnp.int32, (nrows, 1), 0)
    ok = (deg > 0) & (rows < N)
    return jnp.where(ok, lax.rsqrt(jnp.maximum(deg, 1.0)), 0.0)


def _pre_body(x_ref, w_ref, dq_ref, o_ref):
    m = pl.program_id(0)
    nsrc = _norms(dq_ref[...], 0, m * BM, BM)
    h = jnp.dot(x_ref[...], w_ref[...], preferred_element_type=jnp.float32)
    o_ref[...] = h * nsrc


def _tc_pre(x_pad, W, degq):
    return pl.pallas_call(
        _pre_body,
        grid=(NPAD // BM,),
        in_specs=[
            pl.BlockSpec((BM, D), lambda m: (m, 0)),
            pl.BlockSpec((D, D), lambda m: (0, 0)),
            pl.BlockSpec((BM, 4), lambda m: (m, 0)),
        ],
        out_specs=pl.BlockSpec((BM, D), lambda m: (m, 0)),
        out_shape=jax.ShapeDtypeStruct((NPAD, D), jnp.float32),
    )(x_pad, W, degq)


def _layer_h(p_ref, dq, b_ref, g_ref, be_ref, row0, nrows):
    ndst = _norms(dq, 1, row0, nrows)
    agg = (p_ref[0] + p_ref[1]) * ndst
    a1 = jnp.maximum(agg + b_ref[...], 0.0)
    gs = g_ref[...] * lax.rsqrt(jnp.float32(1.0 + EPS))
    return jnp.maximum(a1 * gs + be_ref[...], 0.0)


def _mid_body(p_ref, dq_ref, b_ref, g_ref, be_ref, w_ref,
              h_ref, hp_ref):
    m = pl.program_id(0)
    dq = dq_ref[...]
    h = _layer_h(p_ref, dq, b_ref, g_ref, be_ref, m * BM, BM)
    h_ref[...] = h
    nsrc = _norms(dq, 0, m * BM, BM)
    hp_ref[...] = jnp.dot(h, w_ref[...],
                          preferred_element_type=jnp.float32) * nsrc


def _tc_mid(partials, degq, b, g, be, Wnext):
    return pl.pallas_call(
        _mid_body,
        grid=(NPAD // BM,),
        in_specs=[
            pl.BlockSpec((NC, BM, D), lambda m: (0, m, 0)),
            pl.BlockSpec((BM, 4), lambda m: (m, 0)),
            pl.BlockSpec((1, D), lambda m: (0, 0)),
            pl.BlockSpec((1, D), lambda m: (0, 0)),
            pl.BlockSpec((1, D), lambda m: (0, 0)),
            pl.BlockSpec((D, D), lambda m: (0, 0)),
        ],
        out_specs=[
            pl.BlockSpec((BM, D), lambda m: (m, 0)),
            pl.BlockSpec((BM, D), lambda m: (m, 0)),
        ],
        out_shape=[
            jax.ShapeDtypeStruct((NPAD, D), jnp.float32),
            jax.ShapeDtypeStruct((NPAD, D), jnp.float32),
        ],
    )(partials, degq, b, g, be, Wnext)


def _last_body(p_ref, dq_ref, b_ref, g_ref, be_ref, h0_ref, h1_ref,
               lw_ref, lb_ref, o_ref):
    m = pl.program_id(0)
    h2 = _layer_h(p_ref, dq_ref[...], b_ref, g_ref, be_ref, m * BML, BML)
    jk = jnp.maximum(jnp.maximum(h0_ref[...], h1_ref[...]), h2)
    o_ref[...] = jnp.maximum(
        jnp.dot(jk, lw_ref[...], preferred_element_type=jnp.float32)
        + lb_ref[...], 0.0)


def _tc_last(partials, degq, b, g, be, h0, h1, lin1_W, lin1_b):
    return pl.pallas_call(
        _last_body,
        grid=(N // BML,),
        in_specs=[
            pl.BlockSpec((NC, BML, D), lambda m: (0, m, 0)),
            pl.BlockSpec((BML, 4), lambda m: (m, 0)),
            pl.BlockSpec((1, D), lambda m: (0, 0)),
            pl.BlockSpec((1, D), lambda m: (0, 0)),
            pl.BlockSpec((1, D), lambda m: (0, 0)),
            pl.BlockSpec((BML, D), lambda m: (m, 0)),
            pl.BlockSpec((BML, D), lambda m: (m, 0)),
            pl.BlockSpec((D, D), lambda m: (0, 0)),
            pl.BlockSpec((1, D), lambda m: (0, 0)),
        ],
        out_specs=pl.BlockSpec((BML, D), lambda m: (m, 0)),
        out_shape=jax.ShapeDtypeStruct((N, D), jnp.float32),
    )(partials, degq, b, g, be, h0, h1, lin1_W, lin1_b)


def kernel(adj_t, x, W0, b0, g0, be0, W1, b1, g1, be1, W2, b2, g2, be2,
           lin1_W, lin1_b):
    pad = EPAD - E
    padv = jnp.full((pad,), N, jnp.int32)
    srcp = jnp.concatenate([adj_t[0], padv])
    dstp = jnp.concatenate([adj_t[1], padv])
    srcr = srcp.reshape(NS, CF + CS, K)
    dstr = dstp.reshape(NS, CF + CS, K)
    srcf, srcs = srcr[:, :CF], srcr[:, CF:]
    dstf, dsts = dstr[:, :CF], dstr[:, CF:]
    x_pad = jnp.pad(x, ((0, NPAD - N), (0, 0)))
    zeros = jnp.zeros((NPAD, D), jnp.float32)
    il3 = jnp.stack([srcp.reshape(NT, EPT), dstp.reshape(NT, EPT)],
                    axis=-1).reshape(NT, 2 * NCH, K)
    eye2 = jnp.eye(2, D, dtype=jnp.float32)
    e01 = jnp.tile(eye2, (K // 2, 1))

    degp = _build_deg()(il3, e01, zeros)
    degq = degp[:, :, :2].transpose(1, 0, 2).reshape(NPAD, NC * 2)

    b = [b0.reshape(1, D), b1.reshape(1, D), b2.reshape(1, D)]
    g = [g0.reshape(1, D), g1.reshape(1, D), g2.reshape(1, D)]
    be = [be0.reshape(1, D), be1.reshape(1, D), be2.reshape(1, D)]

    hp = _tc_pre(x_pad, W0, degq)
    p0 = _build_agg()(hp, srcf, dstf, srcs, dsts, zeros)
    h0, hp = _tc_mid(p0, degq, b[0], g[0], be[0], W1)
    p1 = _build_agg()(hp, srcf, dstf, srcs, dsts, zeros)
    h1, hp = _tc_mid(p1, degq, b[1], g[1], be[1], W2)
    p2 = _build_agg()(hp, srcf, dstf, srcs, dsts, zeros)
    out = _tc_last(p2, degq, b[2], g[2], be[2], h0, h1,
                   lin1_W, lin1_b.reshape(1, D))
    return out
